# baseline (device time: 30527 ns/iter reference)
import jax
import jax.numpy as jnp
from jax import lax
from jax.experimental import pallas as pl
from jax.experimental.pallas import tpu as pltpu

N_DEV = 16
P = 4
Z = 4
N_DIR = 2
GA = 3
COLW = 128
A_N = N_DIR * GA * COLW


def kernel(x, w_mat):
    m, k = x.shape
    _, n = w_mat.shape
    m_chunk = m // N_DEV
    blk_rows = m // P
    b_n = n - A_N

    def body(x_ref, w_ref, out_ref, xp_ref, pacc_ref, bacc_ref,
             a1_buf, a2_buf, a_r, b1_buf, b2_buf, b_r,
             a1_send, a1_recv, a2_send, a2_recv,
             b1_send, b1_recv, b2_send, b2_recv):
        my = lax.axis_index("i")
        q = lax.rem(my, P)
        p = lax.div(my, P)
        plane_r = p * P + lax.rem(q + 1, P)
        plane_l = p * P + lax.rem(q + 3, P)
        col_u = lax.rem(p + 1, Z) * P + q
        col_d = lax.rem(p + 3, Z) * P + q

        barrier_sem = pltpu.get_barrier_semaphore()
        for nbr in (plane_l, plane_r, col_u, col_d):
            pl.semaphore_signal(
                barrier_sem, inc=1,
                device_id=(nbr,), device_id_type=pl.DeviceIdType.MESH,
            )

        for qb in range(P):
            for t in range(Z):
                xp_ref[qb * blk_rows + t * m_chunk:
                       qb * blk_rows + (t + 1) * m_chunk, :] = (
                    x_ref[(Z * t + qb) * m_chunk:
                          (Z * t + qb + 1) * m_chunk, :]
                )

        def compute_qblock(qb):
            pacc_ref[pl.ds(qb * blk_rows, blk_rows), :] = jnp.dot(
                xp_ref[pl.ds(qb * blk_rows, blk_rows), :],
                w_ref[:, 0:A_N], preferred_element_type=jnp.float32,
            )

        def compute_bblock(t):
            bacc_ref[pl.ds(t * blk_rows, blk_rows), :] = jnp.dot(
                x_ref[pl.ds(t * blk_rows, blk_rows), :],
                w_ref[:, A_N:n], preferred_element_type=jnp.float32,
            )

        compute_bblock(lax.rem(p + 3, Z))
        compute_bblock(lax.rem(p + 1, Z))
        compute_qblock(lax.rem(q + 3, P))
        compute_qblock(lax.rem(q + 1, P))

        pl.semaphore_wait(barrier_sem, 4)

        sa = [(d, g) for d in range(N_DIR) for g in range(GA)]
        sb = list(range(N_DIR))

        def acol0(d, g):
            return (d * GA + g) * COLW

        def bcol0(d):
            return A_N + d * COLW

        def qblock(qb, d, g):
            return pacc_ref[pl.ds(qb * blk_rows, blk_rows),
                            acol0(d, g):acol0(d, g) + COLW]

        def bblock(t, d):
            return bacc_ref[pl.ds(t * blk_rows, blk_rows),
                            d * COLW:(d + 1) * COLW]

        def a_group(d, g, t):
            return a_r[d, g, pl.ds(t * m_chunk, m_chunk), :]

        def b_group(d, j):
            return b_r[d, pl.ds(j * m_chunk, m_chunk), :]

        def plane_send_idx(d, s):
            return lax.rem(q + 3 - s, P) if d == 0 else lax.rem(q + s + 1, P)

        def col_send_idx(d, s):
            return lax.rem(p + 3 - s, Z) if d == 0 else lax.rem(p + s + 1, Z)

        def make_rdma(buf, send, recv, idx, s, to_dev):
            return pltpu.make_async_remote_copy(
                src_ref=buf.at[idx + (s,)],
                dst_ref=buf.at[idx + (s + 1,)],
                send_sem=send.at[idx + (s,)],
                recv_sem=recv.at[idx + (s,)],
                device_id=(to_dev,),
                device_id_type=pl.DeviceIdType.MESH,
            )

        plane_to = {0: plane_r, 1: plane_l}
        col_to = {0: col_u, 1: col_d}
        rdmas = {}

        def start(key, rdma):
            rdmas[key] = rdma
            rdma.start()

        for d in sb:
            b1_buf[d, 0, :, :] = bblock(col_send_idx(d, 0), d)
            start(("b1", d, 0),
                  make_rdma(b1_buf, b1_send, b1_recv, (d,), 0, col_to[d]))
        for d, g in sa:
            a1_buf[d, g, 0, :, :] = qblock(plane_send_idx(d, 0), d, g)
            start(("a1", d, g, 0),
                  make_rdma(a1_buf, a1_send, a1_recv, (d, g), 0, plane_to[d]))

        compute_bblock(lax.rem(p + 2, Z))
        compute_bblock(p)
        compute_qblock(lax.rem(q + 2, P))
        compute_qblock(q)

        for s in range(1, 3):
            for d in sb:
                rdmas[("b1", d, s - 1)].wait_recv()
                b1_buf[d, s, :, :] = (
                    b1_buf[d, s, :, :] + bblock(col_send_idx(d, s), d)
                )
                start(("b1", d, s),
                      make_rdma(b1_buf, b1_send, b1_recv, (d,), s, col_to[d]))
            for d, g in sa:
                rdmas[("a1", d, g, s - 1)].wait_recv()
                a1_buf[d, g, s, :, :] = (
                    a1_buf[d, g, s, :, :] + qblock(plane_send_idx(d, s), d, g)
                )
                start(("a1", d, g, s),
                      make_rdma(a1_buf, a1_send, a1_recv, (d, g), s,
                                plane_to[d]))

        for d in sb:
            rdmas[("b1", d, 2)].wait_recv()
            j0 = plane_send_idx(d, 0)
            b2_buf[d, 0, :, :] = (
                b1_buf[d, 3, pl.ds(j0 * m_chunk, m_chunk), :]
                + bacc_ref[pl.ds(p * blk_rows + j0 * m_chunk, m_chunk),
                           d * COLW:(d + 1) * COLW]
            )
            start(("b2", d, 0),
                  make_rdma(b2_buf, b2_send, b2_recv, (d,), 0, plane_to[d]))
            b_r[d, :, :] = b1_buf[d, 3, :, :] + bblock(p, d)
        for d, g in sa:
            rdmas[("a1", d, g, 2)].wait_recv()
            t0 = col_send_idx(d, 0)
            a2_buf[d, g, 0, :, :] = (
                a1_buf[d, g, 3, pl.ds(t0 * m_chunk, m_chunk), :]
                + pacc_ref[pl.ds(q * blk_rows + t0 * m_chunk, m_chunk),
                           acol0(d, g):acol0(d, g) + COLW]
            )
            start(("a2", d, g, 0),
                  make_rdma(a2_buf, a2_send, a2_recv, (d, g), 0, col_to[d]))
            a_r[d, g, :, :] = a1_buf[d, g, 3, :, :] + qblock(q, d, g)

        for s in range(1, 3):
            for d in sb:
                rdmas[("b2", d, s - 1)].wait_recv()
                b2_buf[d, s, :, :] = (
                    b2_buf[d, s, :, :] + b_group(d, plane_send_idx(d, s))
                )
                start(("b2", d, s),
                      make_rdma(b2_buf, b2_send, b2_recv, (d,), s,
                                plane_to[d]))
            for d, g in sa:
                rdmas[("a2", d, g, s - 1)].wait_recv()
                a2_buf[d, g, s, :, :] = (
                    a2_buf[d, g, s, :, :] + a_group(d, g, col_send_idx(d, s))
                )
                start(("a2", d, g, s),
                      make_rdma(a2_buf, a2_send, a2_recv, (d, g), s,
                                col_to[d]))

        for d in sb:
            rdmas[("b2", d, 2)].wait_recv()
            out_ref[:, bcol0(d):bcol0(d) + COLW] = jnp.maximum(
                b2_buf[d, 3, :, :] + b_group(d, q), 0.0
            )
        for d, g in sa:
            rdmas[("a2", d, g, 2)].wait_recv()
            out_ref[:, acol0(d, g):acol0(d, g) + COLW] = jnp.maximum(
                a2_buf[d, g, 3, :, :] + a_group(d, g, p), 0.0
            )

        for rdma in rdmas.values():
            rdma.wait_send()

    dma3 = lambda *shape: pltpu.SemaphoreType.DMA(shape)
    return pl.pallas_call(
        body,
        out_shape=jax.ShapeDtypeStruct((m_chunk, n), jnp.float32),
        in_specs=[
            pl.BlockSpec(memory_space=pltpu.VMEM),
            pl.BlockSpec(memory_space=pltpu.VMEM),
        ],
        out_specs=pl.BlockSpec(memory_space=pltpu.VMEM),
        scratch_shapes=[
            pltpu.VMEM((m, k), jnp.float32),
            pltpu.VMEM((m, A_N), jnp.float32),
            pltpu.VMEM((m, b_n), jnp.float32),
            pltpu.VMEM((N_DIR, GA, P, blk_rows, COLW), jnp.float32),
            pltpu.VMEM((N_DIR, GA, Z, m_chunk, COLW), jnp.float32),
            pltpu.VMEM((N_DIR, GA, blk_rows, COLW), jnp.float32),
            pltpu.VMEM((N_DIR, Z, blk_rows, COLW), jnp.float32),
            pltpu.VMEM((N_DIR, P, m_chunk, COLW), jnp.float32),
            pltpu.VMEM((N_DIR, blk_rows, COLW), jnp.float32),
            dma3(N_DIR, GA, P - 1), dma3(N_DIR, GA, P - 1),
            dma3(N_DIR, GA, Z - 1), dma3(N_DIR, GA, Z - 1),
            dma3(N_DIR, Z - 1), dma3(N_DIR, Z - 1),
            dma3(N_DIR, P - 1), dma3(N_DIR, P - 1),
        ],
        compiler_params=pltpu.CompilerParams(collective_id=0),
    )(x, w_mat)
